# Initial kernel scaffold; baseline (speedup 1.0000x reference)
import functools

import jax
import jax.numpy as jnp
from jax import lax
from jax.experimental import pallas as pl
from jax.experimental.pallas import tpu as pltpu
from jax.experimental.pallas import tpu_sc as plsc


def _mm_body(F, x_ref, w_ref, b_ref, xw1_ref, z_ref):
    x = x_ref[...]
    w = w_ref[...]
    xw1_ref[...] = (
        jnp.dot(x, w[:F, :], preferred_element_type=jnp.float32) + b_ref[...]
    )
    z_ref[...] = jnp.dot(x, w[F:, :], preferred_element_type=jnp.float32)


def _phase1(x, W, b2):
    N, F = x.shape
    CH = W.shape[1]
    return pl.pallas_call(
        functools.partial(_mm_body, F),
        out_shape=(
            jax.ShapeDtypeStruct((N, CH), jnp.float32),
            jax.ShapeDtypeStruct((N, CH), jnp.float32),
        ),
    )(x, W, b2)


_B = 128


def _sc_body(
    N, CH, NC, NS, n_chunks, iters, cnt_rows,
    z_hbm, src_hbm, dst_hbm,
    seg_out, cnt_out,
    srcv, dstv, rows, cntl, zbuf, ridx, acc, cacc, sem,
):
    cid = lax.axis_index("c")
    sid = lax.axis_index("s")
    wid = sid * NC + cid
    NW = NC * NS

    zvec = jnp.zeros((16,), jnp.float32)

    for r in range(16):
        for k in range(CH // 16):
            zbuf[r, pl.ds(k * 16, 16)] = zvec

    def _zc(r, _):
        cntl[r, :] = zvec
        return 0

    lax.fori_loop(0, cnt_rows, _zc, 0)

    cshare = cnt_rows // NS
    pltpu.sync_copy(
        cntl.at[pl.ds(sid * cshare, cshare)],
        cacc.at[pl.ds(sid * cshare, cshare)],
    )

    n_zchunk = N // 16

    def _za(i, _):
        ck = i * NS + sid

        @pl.when(ck < n_zchunk)
        def _():
            pltpu.sync_copy(zbuf, acc.at[pl.ds(ck * 16, 16)])

        return 0

    lax.fori_loop(0, (n_zchunk + NS - 1) // NS, _za, 0)

    n_ridx = cnt_rows // _B
    for j in range(n_ridx):
        for k in range(_B // 16):
            ridx[j, pl.ds(k * 16, 16)] = (
                lax.iota(jnp.int32, (16,)) + (j * _B + k * 16)
            )

    plsc.subcore_barrier()

    ones16 = jnp.ones((16,), jnp.float32)

    def _chunk(i, _):
        c = i * NW + wid

        @pl.when(c < n_chunks)
        def _():
            base = c * _B
            pltpu.sync_copy(src_hbm.at[pl.ds(base, _B)], srcv)
            pltpu.sync_copy(dst_hbm.at[pl.ds(base, _B)], dstv)
            pltpu.async_copy(z_hbm.at[srcv], rows, sem).wait()
            pltpu.sync_copy(rows, acc.at[dstv], add=True)
            for k in range(_B // 16):
                d16 = dstv[pl.ds(k * 16, 16)]
                plsc.addupdate_scatter(
                    cntl,
                    [lax.shift_right_logical(d16, 4),
                     lax.bitwise_and(d16, 15)],
                    ones16,
                )

        return 0

    lax.fori_loop(0, iters, _chunk, 0)

    for j in range(n_ridx):
        pltpu.sync_copy(
            cntl.at[pl.ds(j * _B, _B)], cacc.at[ridx.at[j]], add=True
        )

    plsc.subcore_barrier()

    share = N // NS
    r0 = sid * share
    off = 0
    while off < share:
        step = min(_B, share - off)
        pltpu.sync_copy(
            acc.at[pl.ds(r0 + off, step)],
            seg_out.at[pl.ds(cid * N + r0 + off, step)],
        )
        off += step

    pltpu.sync_copy(
        cacc.at[pl.ds(sid * cshare, cshare)],
        cnt_out.at[pl.ds(cid * cnt_rows + sid * cshare, cshare)],
    )


def _phase2(z, src, dst):
    N, CH = z.shape
    E = src.shape[0]
    info = plsc.get_sparse_core_info()
    NC, NS = info.num_cores, info.num_subcores
    NW = NC * NS
    assert E % _B == 0 and N % 16 == 0 and N % NS == 0
    n_chunks = E // _B
    iters = (n_chunks + NW - 1) // NW
    cnt_rows = (N // 16 + _B - 1) // _B * _B
    assert cnt_rows % NS == 0

    mesh = plsc.VectorSubcoreMesh(core_axis_name="c", subcore_axis_name="s")
    body = functools.partial(
        _sc_body, N, CH, NC, NS, n_chunks, iters, cnt_rows
    )
    return pl.kernel(
        body,
        out_type=(
            jax.ShapeDtypeStruct((NC * N, CH), jnp.float32),
            jax.ShapeDtypeStruct((NC * cnt_rows, 16), jnp.float32),
        ),
        mesh=mesh,
        scratch_types=(
            pltpu.VMEM((_B,), jnp.int32),
            pltpu.VMEM((_B,), jnp.int32),
            pltpu.VMEM((_B, CH), jnp.float32),
            pltpu.VMEM((cnt_rows, 16), jnp.float32),
            pltpu.VMEM((16, CH), jnp.float32),
            pltpu.VMEM((cnt_rows // _B, _B), jnp.int32),
            pltpu.VMEM_SHARED((N, CH), jnp.float32),
            pltpu.VMEM_SHARED((cnt_rows, 16), jnp.float32),
            pltpu.SemaphoreType.DMA,
        ),
    )(z, src, dst)


def _fin_body(xw1_ref, seg_ref, cnt_ref, wd_ref, bd_ref, y_ref):
    seg = seg_ref[0] + seg_ref[1]
    cnt = cnt_ref[0] + cnt_ref[1]
    out = xw1_ref[...] + seg / jnp.maximum(cnt, 1.0)
    sq = jnp.sum(out * out, axis=-1, keepdims=True)
    out = out * lax.rsqrt(jnp.maximum(sq, 1e-12))
    out = jnp.maximum(out, 0.0)
    pooled = jnp.sum(out, axis=0, keepdims=True)
    y_ref[...] = (
        jnp.dot(pooled, wd_ref[...], preferred_element_type=jnp.float32)
        + bd_ref[...]
    )


def _phase3(xw1b, seg3, cnt3, Wd, bd2):
    n_out = Wd.shape[1]
    return pl.pallas_call(
        _fin_body,
        out_shape=jax.ShapeDtypeStruct((1, n_out), jnp.float32),
    )(xw1b, seg3, cnt3, Wd, bd2)


def kernel(x, edge_index, W, b, Wd, bd):
    N, F = x.shape
    CH = W.shape[1]
    xw1b, z = _phase1(x, W, b.reshape(1, CH))
    src = edge_index[0]
    dst = edge_index[1]
    seg, cnt = _phase2(z, src, dst)
    NC = seg.shape[0] // N
    seg3 = seg.reshape(NC, N, CH)
    cnt3 = cnt.reshape(NC, -1)[:, :N][..., None]
    y = _phase3(xw1b, seg3, cnt3, Wd, bd.reshape(1, -1))
    return y.reshape(-1)


# speedup vs baseline: 7.3579x; 7.3579x over previous
import functools

import jax
import jax.numpy as jnp
from jax import lax
from jax.experimental import pallas as pl
from jax.experimental.pallas import tpu as pltpu
from jax.experimental.pallas import tpu_sc as plsc


def _mm_body(F, x_ref, w_ref, b_ref, xw1_ref, z_ref):
    x = x_ref[...]
    w = w_ref[...]
    n = x.shape[0]
    xw1_ref[...] = (
        jnp.dot(x, w[:F, :], preferred_element_type=jnp.float32) + b_ref[...]
    )
    z = jnp.dot(x, w[F:, :], preferred_element_type=jnp.float32)
    ch = z.shape[1]
    pad = jnp.zeros((n, 128 - ch - 1), jnp.float32)
    ones = jnp.ones((n, 1), jnp.float32)
    z_ref[...] = jnp.concatenate([z, ones, pad], axis=-1)


def _phase1(x, W, b2):
    N, F = x.shape
    CH = W.shape[1]
    return pl.pallas_call(
        functools.partial(_mm_body, F),
        out_shape=(
            jax.ShapeDtypeStruct((N, CH), jnp.float32),
            jax.ShapeDtypeStruct((N, 128), jnp.float32),
        ),
    )(x, W, b2)


_B = 128


def _sc_body(
    N, NC, NS, n_chunks, iters,
    z_hbm, src_hbm, dst_hbm,
    seg_out,
    srcv, dstv, rows, zbuf, acc, sem,
):
    cid = lax.axis_index("c")
    sid = lax.axis_index("s")
    wid = sid * NC + cid
    NW = NC * NS

    zvec = jnp.zeros((16,), jnp.float32)

    for r in range(16):
        for k in range(8):
            zbuf[r, pl.ds(k * 16, 16)] = zvec

    n_zchunk = N // 16

    def _za(i, _):
        ck = i * NS + sid

        @pl.when(ck < n_zchunk)
        def _():
            pltpu.sync_copy(zbuf, acc.at[pl.ds(ck * 16, 16)])

        return 0

    lax.fori_loop(0, (n_zchunk + NS - 1) // NS, _za, 0)

    plsc.subcore_barrier()

    def _chunk(i, _):
        c = i * NW + wid

        @pl.when(c < n_chunks)
        def _():
            base = c * _B
            pltpu.sync_copy(src_hbm.at[pl.ds(base, _B)], srcv)
            pltpu.sync_copy(dst_hbm.at[pl.ds(base, _B)], dstv)
            pltpu.async_copy(z_hbm.at[srcv], rows, sem).wait()
            pltpu.sync_copy(rows, acc.at[dstv], add=True)

        return 0

    lax.fori_loop(0, iters, _chunk, 0)

    plsc.subcore_barrier()

    def _wb(i, _):
        ck = i * NS + sid

        @pl.when(ck < n_zchunk)
        def _():
            pltpu.sync_copy(
                acc.at[pl.ds(ck * 16, 16)],
                seg_out.at[pl.ds(cid * N + ck * 16, 16)],
            )

        return 0

    lax.fori_loop(0, (n_zchunk + NS - 1) // NS, _wb, 0)


def _phase2(z128, src, dst):
    N = z128.shape[0]
    E = src.shape[0]
    info = plsc.get_sparse_core_info()
    NC, NS = info.num_cores, info.num_subcores
    NW = NC * NS
    assert E % _B == 0 and N % 16 == 0
    n_chunks = E // _B
    iters = (n_chunks + NW - 1) // NW

    mesh = plsc.VectorSubcoreMesh(core_axis_name="c", subcore_axis_name="s")
    body = functools.partial(_sc_body, N, NC, NS, n_chunks, iters)
    return pl.kernel(
        body,
        out_type=jax.ShapeDtypeStruct((NC * N, 128), jnp.float32),
        mesh=mesh,
        scratch_types=(
            pltpu.VMEM((_B,), jnp.int32),
            pltpu.VMEM((_B,), jnp.int32),
            pltpu.VMEM((_B, 128), jnp.float32),
            pltpu.VMEM((16, 128), jnp.float32),
            pltpu.VMEM_SHARED((N, 128), jnp.float32),
            pltpu.SemaphoreType.DMA,
        ),
    )(z128, src, dst)


def _fin_body(CH, xw1_ref, seg_ref, wd_ref, bd_ref, y_ref):
    seg = seg_ref[0] + seg_ref[1]
    cnt = seg[:, CH:CH + 1]
    out = xw1_ref[...] + seg[:, :CH] / jnp.maximum(cnt, 1.0)
    sq = jnp.sum(out * out, axis=-1, keepdims=True)
    out = out * lax.rsqrt(jnp.maximum(sq, 1e-12))
    out = jnp.maximum(out, 0.0)
    pooled = jnp.sum(out, axis=0, keepdims=True)
    y_ref[...] = (
        jnp.dot(pooled, wd_ref[...], preferred_element_type=jnp.float32)
        + bd_ref[...]
    )


def _phase3(xw1b, seg3, Wd, bd2):
    CH, n_out = Wd.shape
    return pl.pallas_call(
        functools.partial(_fin_body, CH),
        out_shape=jax.ShapeDtypeStruct((1, n_out), jnp.float32),
    )(xw1b, seg3, Wd, bd2)


def kernel(x, edge_index, W, b, Wd, bd):
    N, F = x.shape
    CH = W.shape[1]
    xw1b, z128 = _phase1(x, W, b.reshape(1, CH))
    src = edge_index[0]
    dst = edge_index[1]
    seg = _phase2(z128, src, dst)
    NC = seg.shape[0] // N
    seg3 = seg.reshape(NC, N, 128)
    y = _phase3(xw1b, seg3, Wd, bd.reshape(1, -1))
    return y.reshape(-1)


# speedup vs baseline: 16.1115x; 2.1897x over previous
import functools

import jax
import jax.numpy as jnp
from jax import lax
from jax.experimental import pallas as pl
from jax.experimental.pallas import tpu as pltpu
from jax.experimental.pallas import tpu_sc as plsc


_AW = 80


def _mm_body(F, x_ref, w_ref, b_ref, xw1_ref, z_ref):
    x = x_ref[...]
    w = w_ref[...]
    n = x.shape[0]
    xw1_ref[...] = (
        jnp.dot(x, w[:F, :], preferred_element_type=jnp.float32) + b_ref[...]
    )
    z = jnp.dot(x, w[F:, :], preferred_element_type=jnp.float32)
    ch = z.shape[1]
    pad = jnp.zeros((n, _AW - ch - 1), jnp.float32)
    ones = jnp.ones((n, 1), jnp.float32)
    z_ref[...] = jnp.concatenate([z, ones, pad], axis=-1)


def _phase1(x, W, b2):
    N, F = x.shape
    CH = W.shape[1]
    return pl.pallas_call(
        functools.partial(_mm_body, F),
        out_shape=(
            jax.ShapeDtypeStruct((N, CH), jnp.float32),
            jax.ShapeDtypeStruct((N, _AW), jnp.float32),
        ),
    )(x, W, b2)


_B = 128
_NBUF = 3


def _sc_body(
    N, NC, NS, n_chunks, P,
    z_hbm, src_hbm, dst_hbm,
    seg_out,
    srcv, dstv, rows, zbuf, acc, gsems, ssems,
):
    cid = lax.axis_index("c")
    sid = lax.axis_index("s")
    wid = sid * NC + cid

    zvec = jnp.zeros((16,), jnp.float32)

    for r in range(16):
        for k in range(_AW // 16):
            zbuf[r, pl.ds(k * 16, 16)] = zvec

    n_zchunk = N // 16

    def _za(i, _):
        ck = i * NS + sid

        @pl.when(ck < n_zchunk)
        def _():
            pltpu.sync_copy(zbuf, acc.at[pl.ds(ck * 16, 16)])

        return 0

    lax.fori_loop(0, (n_zchunk + NS - 1) // NS, _za, 0)

    base = pl.multiple_of(wid * P, 8)
    pltpu.sync_copy(src_hbm.at[pl.ds(base, P)], srcv)
    pltpu.sync_copy(dst_hbm.at[pl.ds(base, P)], dstv)

    lim = jnp.clip(n_chunks - wid * P, 0, P)

    plsc.subcore_barrier()

    def _gather_start(c, b):
        pltpu.async_copy(z_hbm.at[srcv.at[c]], rows.at[b], gsems.at[b])

    def _gather_wait(c, b):
        pltpu.make_async_copy(
            z_hbm.at[srcv.at[c]], rows.at[b], gsems.at[b]
        ).wait()

    def _scat_start(c, b):
        pltpu.async_copy(rows.at[b], acc.at[dstv.at[c]], ssems.at[b],
                         add=True)

    def _scat_wait(c, b):
        pltpu.make_async_copy(
            rows.at[b], acc.at[dstv.at[c]], ssems.at[b]
        ).wait()

    n_slots = ((P + 1) + _NBUF - 1) // _NBUF * _NBUF

    def _step(j, _):
        for u in range(_NBUF):
            c = j * _NBUF + u
            b = u

            @pl.when(jnp.logical_and(c >= _NBUF, c - _NBUF < lim))
            def _():
                _scat_wait(c - _NBUF, b)

            @pl.when(c < lim)
            def _():
                _gather_start(c, b)

            bp = (u - 1) % _NBUF

            @pl.when(jnp.logical_and(c >= 1, c - 1 < lim))
            def _():
                _gather_wait(c - 1, bp)
                _scat_start(c - 1, bp)

        return 0

    lax.fori_loop(0, n_slots // _NBUF, _step, 0)

    for c in range(n_slots - _NBUF, n_slots - 1):
        @pl.when(c < lim)
        def _():
            _scat_wait(c, c % _NBUF)

    plsc.subcore_barrier()

    def _wb(i, _):
        ck = i * NS + sid

        @pl.when(ck < n_zchunk)
        def _():
            pltpu.sync_copy(
                acc.at[pl.ds(ck * 16, 16)],
                seg_out.at[pl.ds(cid * N + ck * 16, 16)],
            )

        return 0

    lax.fori_loop(0, (n_zchunk + NS - 1) // NS, _wb, 0)


def _phase2(z128, src2d, dst2d, n_chunks):
    N = z128.shape[0]
    info = plsc.get_sparse_core_info()
    NC, NS = info.num_cores, info.num_subcores
    NW = NC * NS
    assert N % 16 == 0
    n_chunks_pad, b = src2d.shape
    assert b == _B
    P = n_chunks_pad // NW
    assert P * NW == n_chunks_pad and P % 8 == 0

    mesh = plsc.VectorSubcoreMesh(core_axis_name="c", subcore_axis_name="s")
    body = functools.partial(_sc_body, N, NC, NS, n_chunks, P)
    return pl.kernel(
        body,
        out_type=jax.ShapeDtypeStruct((NC * N, _AW), jnp.float32),
        mesh=mesh,
        compiler_params=pltpu.CompilerParams(use_tc_tiling_on_sc=False),
        scratch_types=(
            pltpu.VMEM((P, _B), jnp.int32),
            pltpu.VMEM((P, _B), jnp.int32),
            pltpu.VMEM((_NBUF, _B, _AW), jnp.float32),
            pltpu.VMEM((16, _AW), jnp.float32),
            pltpu.VMEM_SHARED((N, _AW), jnp.float32),
            pltpu.SemaphoreType.DMA((_NBUF,)),
            pltpu.SemaphoreType.DMA((_NBUF,)),
        ),
    )(z128, src2d, dst2d)


def _fin_body(CH, xw1_ref, seg_ref, wd_ref, bd_ref, y_ref):
    seg = seg_ref[0] + seg_ref[1]
    cnt = seg[:, CH:CH + 1]
    out = xw1_ref[...] + seg[:, :CH] / jnp.maximum(cnt, 1.0)
    sq = jnp.sum(out * out, axis=-1, keepdims=True)
    out = out * lax.rsqrt(jnp.maximum(sq, 1e-12))
    out = jnp.maximum(out, 0.0)
    pooled = jnp.sum(out, axis=0, keepdims=True)
    y_ref[...] = (
        jnp.dot(pooled, wd_ref[...], preferred_element_type=jnp.float32)
        + bd_ref[...]
    )


def _phase3(xw1b, seg3, Wd, bd2):
    CH, n_out = Wd.shape
    return pl.pallas_call(
        functools.partial(_fin_body, CH),
        out_shape=jax.ShapeDtypeStruct((1, n_out), jnp.float32),
    )(xw1b, seg3, Wd, bd2)


def kernel(x, edge_index, W, b, Wd, bd):
    N, F = x.shape
    CH = W.shape[1]
    xw1b, z128 = _phase1(x, W, b.reshape(1, CH))

    E = edge_index.shape[1]
    assert E % _B == 0
    info = plsc.get_sparse_core_info()
    NW = info.num_cores * info.num_subcores
    n_chunks = E // _B
    P = -(-n_chunks // NW)
    P = (P + 7) // 8 * 8
    e_pad = NW * P * _B - E
    ei = jnp.concatenate(
        [edge_index, jnp.zeros((2, e_pad), edge_index.dtype)], axis=1
    )
    src2d = ei[0].reshape(-1, _B)
    dst2d = ei[1].reshape(-1, _B)

    seg = _phase2(z128, src2d, dst2d, n_chunks)
    NC = seg.shape[0] // N
    seg3 = seg.reshape(NC, N, _AW)
    y = _phase3(xw1b, seg3, Wd, bd.reshape(1, -1))
    return y.reshape(-1)


# speedup vs baseline: 18.0114x; 1.1179x over previous
import functools

import jax
import jax.numpy as jnp
from jax import lax
from jax.experimental import pallas as pl
from jax.experimental.pallas import tpu as pltpu
from jax.experimental.pallas import tpu_sc as plsc


_AW = 80


def _mm_body(F, x_ref, w_ref, b_ref, xw1_ref, z_ref):
    x = x_ref[...]
    w = w_ref[...]
    n = x.shape[0]
    xw1_ref[...] = (
        jnp.dot(x, w[:F, :], preferred_element_type=jnp.float32) + b_ref[...]
    )
    z = jnp.dot(x, w[F:, :], preferred_element_type=jnp.float32)
    ch = z.shape[1]
    pad = jnp.zeros((n, _AW - ch - 1), jnp.float32)
    ones = jnp.ones((n, 1), jnp.float32)
    z_ref[...] = jnp.concatenate([z, ones, pad], axis=-1)


def _phase1(x, W, b2):
    N, F = x.shape
    CH = W.shape[1]
    return pl.pallas_call(
        functools.partial(_mm_body, F),
        out_shape=(
            jax.ShapeDtypeStruct((N, CH), jnp.float32),
            jax.ShapeDtypeStruct((N, _AW), jnp.float32),
        ),
    )(x, W, b2)


_B = 128
_NBUF = 4


def _sc_body(
    N, NC, NS, n_chunks, P,
    z_hbm, src_hbm, dst_hbm,
    seg_out,
    srcv, dstv, rows, zbuf, acc, gsems, ssems,
):
    cid = lax.axis_index("c")
    sid = lax.axis_index("s")
    wid = sid * NC + cid

    zvec = jnp.zeros((16,), jnp.float32)

    base = pl.multiple_of(wid * P, 8)
    pltpu.async_copy(src_hbm.at[pl.ds(base, P)], srcv, gsems.at[0])
    pltpu.async_copy(dst_hbm.at[pl.ds(base, P)], dstv, gsems.at[1])

    for r in range(128):
        for k in range(_AW // 16):
            zbuf[r, pl.ds(k * 16, 16)] = zvec

    n_zfull = N // 128
    z_tail = N - n_zfull * 128
    n_zchunk = n_zfull + (1 if z_tail else 0)

    def _zstart(i, _):
        ck = i * NS + sid

        @pl.when(ck < n_zfull)
        def _():
            pltpu.async_copy(zbuf, acc.at[pl.ds(ck * 128, 128)],
                             ssems.at[0])

        if z_tail:
            @pl.when(ck == n_zfull)
            def _():
                pltpu.async_copy(zbuf.at[pl.ds(0, z_tail)],
                                 acc.at[pl.ds(n_zfull * 128, z_tail)],
                                 ssems.at[0])

        return 0

    def _zwait(i, _):
        ck = i * NS + sid

        @pl.when(ck < n_zfull)
        def _():
            pltpu.make_async_copy(
                zbuf, acc.at[pl.ds(ck * 128, 128)], ssems.at[0]
            ).wait()

        if z_tail:
            @pl.when(ck == n_zfull)
            def _():
                pltpu.make_async_copy(
                    zbuf.at[pl.ds(0, z_tail)],
                    acc.at[pl.ds(n_zfull * 128, z_tail)], ssems.at[0]
                ).wait()

        return 0

    z_iters = (n_zchunk + NS - 1) // NS
    lax.fori_loop(0, z_iters, _zstart, 0)
    lax.fori_loop(0, z_iters, _zwait, 0)

    pltpu.make_async_copy(src_hbm.at[pl.ds(base, P)], srcv,
                          gsems.at[0]).wait()
    pltpu.make_async_copy(dst_hbm.at[pl.ds(base, P)], dstv,
                          gsems.at[1]).wait()

    lim = jnp.clip(n_chunks - wid * P, 0, P)

    plsc.subcore_barrier()

    def _gather_start(c, b):
        pltpu.async_copy(z_hbm.at[srcv.at[c]], rows.at[b], gsems.at[b])

    def _gather_wait(c, b):
        pltpu.make_async_copy(
            z_hbm.at[srcv.at[c]], rows.at[b], gsems.at[b]
        ).wait()

    def _scat_start(c, b):
        pltpu.async_copy(rows.at[b], acc.at[dstv.at[c]], ssems.at[b],
                         add=True)

    def _scat_wait(c, b):
        pltpu.make_async_copy(
            rows.at[b], acc.at[dstv.at[c]], ssems.at[b]
        ).wait()

    n_slots = ((P + 1) + _NBUF - 1) // _NBUF * _NBUF

    def _step(j, _):
        for u in range(_NBUF):
            c = j * _NBUF + u
            b = u

            @pl.when(jnp.logical_and(c >= _NBUF, c - _NBUF < lim))
            def _():
                _scat_wait(c - _NBUF, b)

            @pl.when(c < lim)
            def _():
                _gather_start(c, b)

            bp = (u - 1) % _NBUF

            @pl.when(jnp.logical_and(c >= 1, c - 1 < lim))
            def _():
                _gather_wait(c - 1, bp)
                _scat_start(c - 1, bp)

        return 0

    lax.fori_loop(0, n_slots // _NBUF, _step, 0)

    for c in range(n_slots - _NBUF, n_slots - 1):
        @pl.when(c < lim)
        def _():
            _scat_wait(c, c % _NBUF)

    plsc.subcore_barrier()

    def _wstart(i, _):
        ck = i * NS + sid

        @pl.when(ck < n_zfull)
        def _():
            pltpu.async_copy(
                acc.at[pl.ds(ck * 128, 128)],
                seg_out.at[pl.ds(cid * N + ck * 128, 128)], ssems.at[0])

        if z_tail:
            @pl.when(ck == n_zfull)
            def _():
                pltpu.async_copy(
                    acc.at[pl.ds(n_zfull * 128, z_tail)],
                    seg_out.at[pl.ds(cid * N + n_zfull * 128, z_tail)],
                    ssems.at[0])

        return 0

    def _wwait(i, _):
        ck = i * NS + sid

        @pl.when(ck < n_zfull)
        def _():
            pltpu.make_async_copy(
                acc.at[pl.ds(ck * 128, 128)],
                seg_out.at[pl.ds(cid * N + ck * 128, 128)], ssems.at[0]
            ).wait()

        if z_tail:
            @pl.when(ck == n_zfull)
            def _():
                pltpu.make_async_copy(
                    acc.at[pl.ds(n_zfull * 128, z_tail)],
                    seg_out.at[pl.ds(cid * N + n_zfull * 128, z_tail)],
                    ssems.at[0]
                ).wait()

        return 0

    lax.fori_loop(0, z_iters, _wstart, 0)
    lax.fori_loop(0, z_iters, _wwait, 0)


def _phase2(z128, src2d, dst2d, n_chunks):
    N = z128.shape[0]
    info = plsc.get_sparse_core_info()
    NC, NS = info.num_cores, info.num_subcores
    NW = NC * NS
    assert N % 16 == 0
    n_chunks_pad, b = src2d.shape
    assert b == _B
    P = n_chunks_pad // NW
    assert P * NW == n_chunks_pad and P % 8 == 0

    mesh = plsc.VectorSubcoreMesh(core_axis_name="c", subcore_axis_name="s")
    body = functools.partial(_sc_body, N, NC, NS, n_chunks, P)
    return pl.kernel(
        body,
        out_type=jax.ShapeDtypeStruct((NC * N, _AW), jnp.float32),
        mesh=mesh,
        compiler_params=pltpu.CompilerParams(use_tc_tiling_on_sc=False),
        scratch_types=(
            pltpu.VMEM((P, _B), jnp.int32),
            pltpu.VMEM((P, _B), jnp.int32),
            pltpu.VMEM((_NBUF, _B, _AW), jnp.float32),
            pltpu.VMEM((128, _AW), jnp.float32),
            pltpu.VMEM_SHARED((N, _AW), jnp.float32),
            pltpu.SemaphoreType.DMA((_NBUF,)),
            pltpu.SemaphoreType.DMA((_NBUF,)),
        ),
    )(z128, src2d, dst2d)


def _fin_body(CH, xw1_ref, seg_ref, wd_ref, bd_ref, y_ref):
    seg = seg_ref[0] + seg_ref[1]
    cnt = seg[:, CH:CH + 1]
    out = xw1_ref[...] + seg[:, :CH] / jnp.maximum(cnt, 1.0)
    sq = jnp.sum(out * out, axis=-1, keepdims=True)
    out = out * lax.rsqrt(jnp.maximum(sq, 1e-12))
    out = jnp.maximum(out, 0.0)
    pooled = jnp.sum(out, axis=0, keepdims=True)
    y_ref[...] = (
        jnp.dot(pooled, wd_ref[...], preferred_element_type=jnp.float32)
        + bd_ref[...]
    )


def _phase3(xw1b, seg3, Wd, bd2):
    CH, n_out = Wd.shape
    return pl.pallas_call(
        functools.partial(_fin_body, CH),
        out_shape=jax.ShapeDtypeStruct((1, n_out), jnp.float32),
    )(xw1b, seg3, Wd, bd2)


def kernel(x, edge_index, W, b, Wd, bd):
    N, F = x.shape
    CH = W.shape[1]
    xw1b, z128 = _phase1(x, W, b.reshape(1, CH))

    E = edge_index.shape[1]
    assert E % _B == 0
    info = plsc.get_sparse_core_info()
    NW = info.num_cores * info.num_subcores
    n_chunks = E // _B
    P = -(-n_chunks // NW)
    P = (P + 7) // 8 * 8
    e_pad = NW * P * _B - E
    ei = jnp.concatenate(
        [edge_index, jnp.zeros((2, e_pad), edge_index.dtype)], axis=1
    )
    src2d = ei[0].reshape(-1, _B)
    dst2d = ei[1].reshape(-1, _B)

    seg = _phase2(z128, src2d, dst2d, n_chunks)
    NC = seg.shape[0] // N
    seg3 = seg.reshape(NC, N, _AW)
    y = _phase3(xw1b, seg3, Wd, bd.reshape(1, -1))
    return y.reshape(-1)
